# baseline (speedup 1.0000x reference)
import functools
import math

import jax
import jax.numpy as jnp
from jax import lax
from jax.experimental import pallas as pl
from jax.experimental.pallas import tpu as pltpu
from jax.experimental.pallas import tpu_sc as plsc

RANK = 32
K1 = 32
B = 4096
NUM_TRAIN = 1000000
BATCH_SIZE = 128
SIGMA = 1.0
C0 = -0.5 * math.log(2.0 * math.pi * SIGMA ** 2)

NW = 32
B_PER_W = B // NW


@functools.cache
def _make_sc_gather():
    @functools.partial(
        pl.kernel,
        mesh=plsc.VectorSubcoreMesh(core_axis_name="c", subcore_axis_name="s"),
        out_type=jax.ShapeDtypeStruct((6 * B, RANK), jnp.float32),
        scratch_types=[
            pltpu.VMEM((B_PER_W,), jnp.int32),
            pltpu.VMEM((B_PER_W, RANK), jnp.float32),
            pltpu.SemaphoreType.DMA,
        ],
        compiler_params=pltpu.CompilerParams(use_tc_tiling_on_sc=False),
    )
    def _sc_gather(idx_hbm, m0, m1, m2, c0, c1, c2, out_hbm, idx_v, rows_v, sem):
        wid = lax.axis_index("s") * 2 + lax.axis_index("c")
        base = wid * B_PER_W
        tables = [m0, m1, m2, c0, c1, c2]
        for t in range(6):
            d = t % 3
            pltpu.sync_copy(idx_hbm.at[pl.ds(d * B + base, B_PER_W)], idx_v)
            pltpu.async_copy(tables[t].at[idx_v], rows_v, sem).wait()
            pltpu.sync_copy(rows_v, out_hbm.at[pl.ds(t * B + base, B_PER_W)])

    return _sc_gather


BB = 256
NBLK = B // BB


def _tc_body(eps_ref, rows_ref, ys_ref, out_ref):
    i = pl.program_id(0)

    @pl.when(i == 0)
    def _init():
        out_ref[...] = jnp.full((1, 1), -(NUM_TRAIN / BATCH_SIZE) * B * C0,
                                dtype=jnp.float32)

    parts = []
    for t in range(6):
        r = rows_ref[t]
        parts.append(jnp.concatenate([r, r, r, r], axis=-1))

    P = None
    for d in range(3):
        m_t = parts[d][:, None, :]
        l_t = parts[3 + d][:, None, :]
        s = m_t + l_t * eps_ref[d]
        P = s if P is None else P * s

    P2 = P.reshape(BB * 8, 128)
    ia = lax.broadcasted_iota(jnp.int32, (128, 128), 0) // 32
    ib = lax.broadcasted_iota(jnp.int32, (128, 128), 1) // 32
    seg = (ia == ib).astype(jnp.float32)
    R = jnp.dot(P2, seg, preferred_element_type=jnp.float32,
                precision=lax.Precision.HIGHEST)

    y = ys_ref[...]
    yb = jnp.broadcast_to(y[:, :, None], (BB, 8, 1)).reshape(BB * 8, 1)
    dlt = yb - R
    d2 = jnp.sum(dlt * dlt)

    kl = jnp.float32(0.0)
    for d in range(3):
        m = rows_ref[d]
        L = rows_ref[3 + d]
        l2 = L * L
        kl = kl + jnp.sum(l2 + m * m - 1.0 - jnp.log(l2 + 1e-8))

    upd = (NUM_TRAIN / BATCH_SIZE) * d2 / (32.0 * 2.0 * K1) \
        + (0.5 / BATCH_SIZE) * kl
    out_ref[...] += jnp.full((1, 1), 1.0, dtype=jnp.float32) * upd


_tc_call = pl.pallas_call(
    _tc_body,
    grid=(NBLK,),
    in_specs=[
        pl.BlockSpec((3, BB, 8, 128), lambda i: (0, i, 0, 0)),
        pl.BlockSpec((6, BB, RANK), lambda i: (0, i, 0)),
        pl.BlockSpec((BB, 1), lambda i: (i, 0)),
    ],
    out_specs=pl.BlockSpec((1, 1), lambda i: (0, 0)),
    out_shape=jax.ShapeDtypeStruct((1, 1), jnp.float32),
    compiler_params=pltpu.CompilerParams(
        dimension_semantics=("arbitrary",),
    ),
)


def kernel(idx, ys, eps, mean0, mean1, mean2, chol0, chol1, chol2):
    idx_flat = idx.T.reshape(-1)
    rows = _make_sc_gather()(idx_flat, mean0, mean1, mean2, chol0, chol1, chol2)
    eps_r = eps.reshape(3, B, K1 * RANK // 128, 128)
    out = _tc_call(eps_r, rows.reshape(6, B, RANK), ys.reshape(B, 1))
    return out.reshape(())
